# baseline (speedup 1.0000x reference)
import functools

import jax
import jax.numpy as jnp
from jax import lax
from jax.experimental import pallas as pl
from jax.experimental.pallas import tpu as pltpu
from jax.experimental.pallas import tpu_sc as plsc

N_CLASS = 8
_NTILES = 32


def _point_sample(feat, points):
    B, C, H, W = feat.shape
    x = points[..., 0] * (W - 1)
    y = points[..., 1] * (H - 1)
    x0f = jnp.floor(x); y0f = jnp.floor(y)
    wx = x - x0f; wy = y - y0f
    x0 = jnp.clip(x0f, 0, W - 1).astype(jnp.int32)
    x1 = jnp.clip(x0f + 1, 0, W - 1).astype(jnp.int32)
    y0 = jnp.clip(y0f, 0, H - 1).astype(jnp.int32)
    y1 = jnp.clip(y0f + 1, 0, H - 1).astype(jnp.int32)
    b = jnp.arange(B)[:, None]
    f00 = feat[b, :, y0, x0]
    f01 = feat[b, :, y0, x1]
    f10 = feat[b, :, y1, x0]
    f11 = feat[b, :, y1, x1]
    wxe = wx[..., None]; wye = wy[..., None]
    out = f00 * (1 - wxe) * (1 - wye) + f01 * wxe * (1 - wye) \
        + f10 * (1 - wxe) * wye + f11 * wxe * wye
    return jnp.transpose(out, (0, 2, 1))


def _upsample2x(feat):
    B, C, H, W = feat.shape
    ys = jnp.linspace(0.0, H - 1.0, 2 * H)
    xs = jnp.linspace(0.0, W - 1.0, 2 * W)
    y0f = jnp.floor(ys); wy = ys - y0f
    x0f = jnp.floor(xs); wx = xs - x0f
    y0 = y0f.astype(jnp.int32); y1 = jnp.minimum(y0 + 1, H - 1)
    x0 = x0f.astype(jnp.int32); x1 = jnp.minimum(x0 + 1, W - 1)
    rows = feat[:, :, y0, :] * (1 - wy)[None, None, :, None] \
        + feat[:, :, y1, :] * wy[None, None, :, None]
    out = rows[:, :, :, x0] * (1 - wx)[None, None, None, :] \
        + rows[:, :, :, x1] * wx[None, None, None, :]
    return out


def _sampling_points(mask, N, key1, key2, k=3, beta=0.75):
    B = mask.shape[0]
    over = jax.random.uniform(key1, (B, k * N, 2), dtype=jnp.float32)
    vals = _point_sample(mask, over)
    t = jax.lax.top_k(jnp.transpose(vals, (0, 2, 1)), 2)[0]
    unc = t[..., 1] - t[..., 0]
    n_imp = int(beta * N)
    idx = jax.lax.top_k(unc, n_imp)[1]
    imp = jnp.take_along_axis(over, idx[..., None], axis=1)
    cov = jax.random.uniform(key2, (B, N - n_imp, 2), dtype=jnp.float32)
    return jnp.concatenate([imp, cov], axis=1)


@functools.partial(jax.jit, static_argnames=("C", "H", "W", "N", "c_chunk"))
def _sc_point_sample(feat_flat, xs, ys, *, C, H, W, N, c_chunk):
    HW = H * W
    M = 2 * N
    c_per_tile = C // _NTILES
    rounds = c_per_tile // c_chunk
    mesh = plsc.VectorSubcoreMesh(core_axis_name="c", subcore_axis_name="s")

    @functools.partial(
        pl.kernel, mesh=mesh,
        compiler_params=pltpu.CompilerParams(needs_layout_passes=False),
        out_type=jax.ShapeDtypeStruct((C, M), jnp.float32),
        scratch_types=[
            pltpu.VMEM((2 * c_chunk * HW,), jnp.float32),
            pltpu.VMEM((c_per_tile, M), jnp.float32),
            pltpu.VMEM((M,), jnp.float32),
            pltpu.VMEM((M,), jnp.float32),
        ],
    )
    def sampler(feat_hbm, xs_hbm, ys_hbm, out_hbm, chunk_v, out_v, xs_v, ys_v):
        wid = lax.axis_index("s") * 2 + lax.axis_index("c")
        c_base = wid * c_per_tile
        pltpu.sync_copy(xs_hbm, xs_v)
        pltpu.sync_copy(ys_hbm, ys_v)
        lanes = lax.iota(jnp.int32, 16)
        for r in range(rounds):
            c0 = c_base + r * c_chunk
            pltpu.sync_copy(feat_hbm.at[pl.ds(c0 * HW, c_chunk * HW)],
                            chunk_v.at[pl.ds(0, c_chunk * HW)])
            pltpu.sync_copy(feat_hbm.at[pl.ds((C + c0) * HW, c_chunk * HW)],
                            chunk_v.at[pl.ds(c_chunk * HW, c_chunk * HW)])
            for b in range(2):
                boff = b * (c_chunk * HW)

                def body(g, _, boff=boff, b=b, r=r):
                    p0 = b * N + g * 16
                    xv = xs_v[pl.ds(p0, 16)]
                    yv = ys_v[pl.ds(p0, 16)]
                    fx = xv * float(W - 1)
                    fy = yv * float(H - 1)
                    ix0 = fx.astype(jnp.int32)
                    iy0 = fy.astype(jnp.int32)
                    wx = fx - ix0.astype(jnp.float32)
                    wy = fy - iy0.astype(jnp.float32)
                    ix1 = jnp.minimum(ix0 + 1, W - 1)
                    iy1 = jnp.minimum(iy0 + 1, H - 1)
                    p00 = iy0 * W + ix0 + boff
                    p01 = iy0 * W + ix1 + boff
                    p10 = iy1 * W + ix0 + boff
                    p11 = iy1 * W + ix1 + boff
                    w00 = (1.0 - wx) * (1.0 - wy)
                    w01 = wx * (1.0 - wy)
                    w10 = (1.0 - wx) * wy
                    w11 = wx * wy
                    pcol = lanes + p0
                    for c in range(c_chunk):
                        off = c * HW
                        v = (plsc.load_gather(chunk_v, [p00 + off]) * w00
                             + plsc.load_gather(chunk_v, [p01 + off]) * w01
                             + plsc.load_gather(chunk_v, [p10 + off]) * w10
                             + plsc.load_gather(chunk_v, [p11 + off]) * w11)
                        cvec = jnp.full((16,), r * c_chunk + c, jnp.int32)
                        plsc.store_scatter(out_v, [cvec, pcol], v)
                    return 0

                lax.fori_loop(0, N // 16, body, 0)
        pltpu.sync_copy(out_v, out_hbm.at[pl.ds(c_base, c_per_tile), :])

    return sampler(feat_flat, xs, ys)


@functools.partial(jax.jit, static_argnames=("C", "H", "W", "N"))
def _sc_point_sample_indirect(feat_flat, xs, ys, *, C, H, W, N):
    HW = H * W
    M = 2 * N
    ppt = max(128, M // _NTILES)
    active = M // ppt
    G = ppt // 16
    nunits = ppt * C * 4
    nchunks = nunits // 128
    mesh = plsc.VectorSubcoreMesh(core_axis_name="c", subcore_axis_name="s")

    @functools.partial(
        pl.kernel, mesh=mesh,
        compiler_params=pltpu.CompilerParams(needs_layout_passes=False),
        out_type=jax.ShapeDtypeStruct((C, M), jnp.float32),
        scratch_types=[
            pltpu.VMEM((nchunks, 128), jnp.int32),
            pltpu.VMEM((nchunks, 128), jnp.float32),
            pltpu.VMEM((C, ppt), jnp.float32),
            pltpu.VMEM((ppt,), jnp.float32),
            pltpu.VMEM((ppt,), jnp.float32),
            pltpu.SemaphoreType.DMA,
        ],
    )
    def sampler(feat_hbm, xs_hbm, ys_hbm, out_hbm,
                idx_v, dest_v, out_v, xs_v, ys_v, sem):
        wid = lax.axis_index("s") * 2 + lax.axis_index("c")

        @pl.when(wid < active)
        def _():
            p_base = wid * ppt
            b = p_base // N
            pltpu.sync_copy(xs_hbm.at[pl.ds(p_base, ppt)], xs_v)
            pltpu.sync_copy(ys_hbm.at[pl.ds(p_base, ppt)], ys_v)
            base_off = b * (C * HW)

            def neighborhood(g):
                xv = xs_v[pl.ds(g * 16, 16)]
                yv = ys_v[pl.ds(g * 16, 16)]
                fx = xv * float(W - 1)
                fy = yv * float(H - 1)
                ix0 = fx.astype(jnp.int32)
                iy0 = fy.astype(jnp.int32)
                wx = fx - ix0.astype(jnp.float32)
                wy = fy - iy0.astype(jnp.float32)
                ix1 = jnp.minimum(ix0 + 1, W - 1)
                iy1 = jnp.minimum(iy0 + 1, H - 1)
                p00 = iy0 * W + ix0
                p01 = iy0 * W + ix1
                p10 = iy1 * W + ix0
                p11 = iy1 * W + ix1
                return (p00, p01, p10, p11), (wx, wy)

            def build(g, _):
                pix, _w = neighborhood(g)
                for c in range(C):
                    coff = base_off + c * HW
                    for k in range(4):
                        o = ((g * C + c) * 4 + k) * 16
                        idx_v[o // 128, pl.ds(o % 128, 16)] = pix[k] + coff
                return 0

            lax.fori_loop(0, G, build, 0)

            def fire(j, _):
                pltpu.async_copy(feat_hbm.at[idx_v.at[j]], dest_v.at[j], sem)
                return 0

            lax.fori_loop(0, nchunks, fire, 0)

            def drain(j, _):
                pltpu.make_async_copy(feat_hbm.at[pl.ds(0, 128)],
                                      dest_v.at[j], sem).wait()
                return 0

            lax.fori_loop(0, nchunks, drain, 0)

            def interp(g, _):
                _pix, (wx, wy) = neighborhood(g)
                w00 = (1.0 - wx) * (1.0 - wy)
                w01 = wx * (1.0 - wy)
                w10 = (1.0 - wx) * wy
                w11 = wx * wy
                for c in range(C):
                    o = ((g * C + c) * 4) * 16
                    r = o // 128
                    f00 = dest_v[r, pl.ds(o % 128, 16)]
                    f01 = dest_v[r, pl.ds(o % 128 + 16, 16)]
                    o2 = o + 32
                    f10 = dest_v[o2 // 128, pl.ds(o2 % 128, 16)]
                    f11 = dest_v[o2 // 128, pl.ds(o2 % 128 + 16, 16)]
                    v = f00 * w00 + f01 * w01 + f10 * w10 + f11 * w11
                    out_v[c, pl.ds(g * 16, 16)] = v
                return 0

            lax.fori_loop(0, G, interp, 0)
            pltpu.sync_copy(out_v, out_hbm.at[:, pl.ds(p_base, ppt)])

    return sampler(feat_flat, xs, ys)


def _mlp_kernel(xm_ref, xf_ref, w1m_ref, w1f_ref, b1_ref, w2_ref, b2_ref,
                w3_ref, b3_ref, wf_ref, bf_ref, o_ref):
    dn = (((1,), (0,)), ((), ()))
    h = lax.dot_general(w1m_ref[...], xm_ref[...], dn,
                        preferred_element_type=jnp.float32)
    h += lax.dot_general(w1f_ref[...], xf_ref[...], dn,
                         preferred_element_type=jnp.float32)
    h = jnp.maximum(h + b1_ref[...], 0.0)
    h = jnp.maximum(lax.dot_general(w2_ref[...], h, dn,
                                    preferred_element_type=jnp.float32)
                    + b2_ref[...], 0.0)
    h = jnp.maximum(lax.dot_general(w3_ref[...], h, dn,
                                    preferred_element_type=jnp.float32)
                    + b3_ref[...], 0.0)
    o_ref[...] = lax.dot_general(wf_ref[...], h, dn,
                                 preferred_element_type=jnp.float32) \
        + bf_ref[...]


@functools.partial(jax.jit, static_argnames=("blk",))
def _mlp_pallas(params, xm, xf, blk=1024):
    W1, b1, W2, b2, W3, b3, Wf, bf = params
    Cf, M = xf.shape
    w1m = W1[:, :N_CLASS]
    w1f = W1[:, N_CLASS:]
    blk = min(blk, M)
    grid = (M // blk,)
    return pl.pallas_call(
        _mlp_kernel,
        grid=grid,
        in_specs=[
            pl.BlockSpec((N_CLASS, blk), lambda i: (0, i)),
            pl.BlockSpec((Cf, blk), lambda i: (0, i)),
            pl.BlockSpec((512, N_CLASS), lambda i: (0, 0)),
            pl.BlockSpec((512, Cf), lambda i: (0, 0)),
            pl.BlockSpec((512, 1), lambda i: (0, 0)),
            pl.BlockSpec((512, 512), lambda i: (0, 0)),
            pl.BlockSpec((512, 1), lambda i: (0, 0)),
            pl.BlockSpec((512, 512), lambda i: (0, 0)),
            pl.BlockSpec((512, 1), lambda i: (0, 0)),
            pl.BlockSpec((N_CLASS, 512), lambda i: (0, 0)),
            pl.BlockSpec((N_CLASS, 1), lambda i: (0, 0)),
        ],
        out_specs=pl.BlockSpec((N_CLASS, blk), lambda i: (0, i)),
        out_shape=jax.ShapeDtypeStruct((N_CLASS, M), jnp.float32),
    )(xm, xf, w1m, w1f, b1[:, None], W2, b2[:, None], W3, b3[:, None],
      Wf, bf[:, None])


def _stage(temp, feat, params, pts, sc_chunk):
    B, C, H, W = feat.shape
    N = pts.shape[1]
    xs = pts[..., 0].reshape(-1)
    ys = pts[..., 1].reshape(-1)
    if sc_chunk is not None:
        xf = _sc_point_sample(feat.reshape(-1), xs, ys,
                              C=C, H=H, W=W, N=N, c_chunk=sc_chunk)
    else:
        xf = _sc_point_sample_indirect(feat.reshape(-1), xs, ys,
                                       C=C, H=H, W=W, N=N)
    xm = jnp.transpose(_point_sample(temp, pts), (1, 0, 2)).reshape(8, B * N)
    out = _mlp_pallas(params, xm, xf)
    return jnp.transpose(out.reshape(N_CLASS, B, N), (1, 0, 2))


def kernel(refine, x0, x1, x2, x3, coarse, p3, p2, p1, p0, pr):
    key = jax.random.key(42)
    ks = jax.random.split(key, 10)
    temp1 = coarse
    pts1 = _sampling_points(jax.nn.softmax(temp1, axis=1), 512, ks[0], ks[1])
    rend1 = _stage(temp1, x3, p3, pts1, 8)
    temp2 = coarse
    pts2 = _sampling_points(jax.nn.softmax(temp2, axis=1), 512, ks[2], ks[3])
    rend2 = _stage(temp2, x2, p2, pts2, 8)
    temp3 = _upsample2x(temp2)
    pts3 = _sampling_points(jax.nn.softmax(temp3, axis=1), 2048, ks[4], ks[5])
    rend3 = _stage(temp3, x1, p1, pts3, 2)
    temp4 = _upsample2x(temp3)
    pts4 = _sampling_points(jax.nn.softmax(temp4, axis=1), 2048, ks[6], ks[7])
    rend4 = _stage(temp4, x0, p0, pts4, None)
    temp5 = _upsample2x(temp4)
    pts5 = _sampling_points(jax.nn.softmax(temp5, axis=1), 2048, ks[8], ks[9])
    rend5 = _stage(temp5, refine, pr, pts5, None)
    return (pts1, rend1, pts2, rend2, pts3, rend3, pts4, rend4, pts5, rend5)


# speedup vs baseline: 1.1379x; 1.1379x over previous
import functools

import jax
import jax.numpy as jnp
from jax import lax
from jax.experimental import pallas as pl
from jax.experimental.pallas import tpu as pltpu
from jax.experimental.pallas import tpu_sc as plsc

N_CLASS = 8
_NTILES = 32


def _point_sample(feat, points):
    B, C, H, W = feat.shape
    x = points[..., 0] * (W - 1)
    y = points[..., 1] * (H - 1)
    x0f = jnp.floor(x); y0f = jnp.floor(y)
    wx = x - x0f; wy = y - y0f
    x0 = jnp.clip(x0f, 0, W - 1).astype(jnp.int32)
    x1 = jnp.clip(x0f + 1, 0, W - 1).astype(jnp.int32)
    y0 = jnp.clip(y0f, 0, H - 1).astype(jnp.int32)
    y1 = jnp.clip(y0f + 1, 0, H - 1).astype(jnp.int32)
    b = jnp.arange(B)[:, None]
    f00 = feat[b, :, y0, x0]
    f01 = feat[b, :, y0, x1]
    f10 = feat[b, :, y1, x0]
    f11 = feat[b, :, y1, x1]
    wxe = wx[..., None]; wye = wy[..., None]
    out = f00 * (1 - wxe) * (1 - wye) + f01 * wxe * (1 - wye) \
        + f10 * (1 - wxe) * wye + f11 * wxe * wye
    return jnp.transpose(out, (0, 2, 1))


def _upsample2x(feat):
    B, C, H, W = feat.shape
    ys = jnp.linspace(0.0, H - 1.0, 2 * H)
    xs = jnp.linspace(0.0, W - 1.0, 2 * W)
    y0f = jnp.floor(ys); wy = ys - y0f
    x0f = jnp.floor(xs); wx = xs - x0f
    y0 = y0f.astype(jnp.int32); y1 = jnp.minimum(y0 + 1, H - 1)
    x0 = x0f.astype(jnp.int32); x1 = jnp.minimum(x0 + 1, W - 1)
    rows = feat[:, :, y0, :] * (1 - wy)[None, None, :, None] \
        + feat[:, :, y1, :] * wy[None, None, :, None]
    out = rows[:, :, :, x0] * (1 - wx)[None, None, None, :] \
        + rows[:, :, :, x1] * wx[None, None, None, :]
    return out


def _sampling_points(mask, N, key1, key2, k=3, beta=0.75):
    B = mask.shape[0]
    over = jax.random.uniform(key1, (B, k * N, 2), dtype=jnp.float32)
    vals = _point_sample(mask, over)
    t = jax.lax.top_k(jnp.transpose(vals, (0, 2, 1)), 2)[0]
    unc = t[..., 1] - t[..., 0]
    n_imp = int(beta * N)
    idx = jax.lax.top_k(unc, n_imp)[1]
    imp = jnp.take_along_axis(over, idx[..., None], axis=1)
    cov = jax.random.uniform(key2, (B, N - n_imp, 2), dtype=jnp.float32)
    return jnp.concatenate([imp, cov], axis=1)


@functools.partial(jax.jit, static_argnames=("C", "H", "W", "N", "c_chunk"))
def _sc_point_sample(feat_flat, xs, ys, *, C, H, W, N, c_chunk):
    HW = H * W
    M = 2 * N
    c_per_tile = C // _NTILES
    rounds = c_per_tile // c_chunk
    mesh = plsc.VectorSubcoreMesh(core_axis_name="c", subcore_axis_name="s")

    @functools.partial(
        pl.kernel, mesh=mesh,
        compiler_params=pltpu.CompilerParams(needs_layout_passes=False),
        out_type=jax.ShapeDtypeStruct((C, M), jnp.float32),
        scratch_types=[
            pltpu.VMEM((2 * c_chunk * HW,), jnp.float32),
            pltpu.VMEM((c_per_tile, M), jnp.float32),
            pltpu.VMEM((M,), jnp.float32),
            pltpu.VMEM((M,), jnp.float32),
        ],
    )
    def sampler(feat_hbm, xs_hbm, ys_hbm, out_hbm, chunk_v, out_v, xs_v, ys_v):
        wid = lax.axis_index("s") * 2 + lax.axis_index("c")
        c_base = wid * c_per_tile
        pltpu.sync_copy(xs_hbm, xs_v)
        pltpu.sync_copy(ys_hbm, ys_v)
        lanes = lax.iota(jnp.int32, 16)
        for r in range(rounds):
            c0 = c_base + r * c_chunk
            pltpu.sync_copy(feat_hbm.at[pl.ds(c0 * HW, c_chunk * HW)],
                            chunk_v.at[pl.ds(0, c_chunk * HW)])
            pltpu.sync_copy(feat_hbm.at[pl.ds((C + c0) * HW, c_chunk * HW)],
                            chunk_v.at[pl.ds(c_chunk * HW, c_chunk * HW)])
            for b in range(2):
                boff = b * (c_chunk * HW)

                def body(g, _, boff=boff, b=b, r=r):
                    p0 = b * N + g * 16
                    xv = xs_v[pl.ds(p0, 16)]
                    yv = ys_v[pl.ds(p0, 16)]
                    fx = xv * float(W - 1)
                    fy = yv * float(H - 1)
                    ix0 = fx.astype(jnp.int32)
                    iy0 = fy.astype(jnp.int32)
                    wx = fx - ix0.astype(jnp.float32)
                    wy = fy - iy0.astype(jnp.float32)
                    ix1 = jnp.minimum(ix0 + 1, W - 1)
                    iy1 = jnp.minimum(iy0 + 1, H - 1)
                    p00 = iy0 * W + ix0 + boff
                    p01 = iy0 * W + ix1 + boff
                    p10 = iy1 * W + ix0 + boff
                    p11 = iy1 * W + ix1 + boff
                    w00 = (1.0 - wx) * (1.0 - wy)
                    w01 = wx * (1.0 - wy)
                    w10 = (1.0 - wx) * wy
                    w11 = wx * wy
                    pcol = lanes + p0
                    for c in range(c_chunk):
                        off = c * HW
                        v = (plsc.load_gather(chunk_v, [p00 + off]) * w00
                             + plsc.load_gather(chunk_v, [p01 + off]) * w01
                             + plsc.load_gather(chunk_v, [p10 + off]) * w10
                             + plsc.load_gather(chunk_v, [p11 + off]) * w11)
                        cvec = jnp.full((16,), r * c_chunk + c, jnp.int32)
                        plsc.store_scatter(out_v, [cvec, pcol], v)
                    return 0

                lax.fori_loop(0, N // 16, body, 0)
        pltpu.sync_copy(out_v, out_hbm.at[pl.ds(c_base, c_per_tile), :])

    return sampler(feat_flat, xs, ys)


@functools.partial(jax.jit, static_argnames=("C", "H", "W", "N", "raw"))
def _sc_point_sample_indirect(feat_flat, xs, ys, *, C, H, W, N, raw=False):
    HW = H * W
    M = 2 * N
    ppt = max(128, M // _NTILES)
    active = M // ppt
    G = ppt // 16
    nunits = ppt * C * 4
    nchunks = nunits // 128
    mesh = plsc.VectorSubcoreMesh(core_axis_name="c", subcore_axis_name="s")
    if raw:
        out_type = jax.ShapeDtypeStruct((4, C, M), jnp.float32)
        out_scratch = pltpu.VMEM((4, C, ppt), jnp.float32)
    else:
        out_type = jax.ShapeDtypeStruct((C, M), jnp.float32)
        out_scratch = pltpu.VMEM((C, ppt), jnp.float32)

    @functools.partial(
        pl.kernel, mesh=mesh,
        compiler_params=pltpu.CompilerParams(needs_layout_passes=False),
        out_type=out_type,
        scratch_types=[
            pltpu.VMEM((nchunks, 128), jnp.int32),
            pltpu.VMEM((nchunks, 128), jnp.float32),
            out_scratch,
            pltpu.VMEM((ppt,), jnp.float32),
            pltpu.VMEM((ppt,), jnp.float32),
            pltpu.SemaphoreType.DMA,
        ],
    )
    def sampler(feat_hbm, xs_hbm, ys_hbm, out_hbm,
                idx_v, dest_v, out_v, xs_v, ys_v, sem):
        wid = lax.axis_index("s") * 2 + lax.axis_index("c")

        @pl.when(wid < active)
        def _():
            p_base = wid * ppt
            b = p_base // N
            pltpu.sync_copy(xs_hbm.at[pl.ds(p_base, ppt)], xs_v)
            pltpu.sync_copy(ys_hbm.at[pl.ds(p_base, ppt)], ys_v)
            base_off = b * (C * HW)

            def neighborhood(g):
                xv = xs_v[pl.ds(g * 16, 16)]
                yv = ys_v[pl.ds(g * 16, 16)]
                fx = xv * float(W - 1)
                fy = yv * float(H - 1)
                ix0 = fx.astype(jnp.int32)
                iy0 = fy.astype(jnp.int32)
                wx = fx - ix0.astype(jnp.float32)
                wy = fy - iy0.astype(jnp.float32)
                ix1 = jnp.minimum(ix0 + 1, W - 1)
                iy1 = jnp.minimum(iy0 + 1, H - 1)
                p00 = iy0 * W + ix0
                p01 = iy0 * W + ix1
                p10 = iy1 * W + ix0
                p11 = iy1 * W + ix1
                return (p00, p01, p10, p11), (wx, wy)

            def build(g, _):
                pix, _w = neighborhood(g)
                for c in range(C):
                    coff = base_off + c * HW
                    for k in range(4):
                        o = ((g * C + c) * 4 + k) * 16
                        idx_v[o // 128, pl.ds(o % 128, 16)] = pix[k] + coff
                return 0

            lax.fori_loop(0, G, build, 0)

            def fire(j, _):
                pltpu.async_copy(feat_hbm.at[idx_v.at[j]], dest_v.at[j], sem)
                return 0

            lax.fori_loop(0, nchunks, fire, 0)

            def drain(j, _):
                pltpu.make_async_copy(feat_hbm.at[pl.ds(0, 128)],
                                      dest_v.at[j], sem).wait()
                return 0

            lax.fori_loop(0, nchunks, drain, 0)

            if raw:
                def reorder(g, _):
                    for c in range(C):
                        for k in range(4):
                            o = ((g * C + c) * 4 + k) * 16
                            v = dest_v[o // 128, pl.ds(o % 128, 16)]
                            out_v[k, c, pl.ds(g * 16, 16)] = v
                    return 0

                lax.fori_loop(0, G, reorder, 0)
                pltpu.sync_copy(out_v, out_hbm.at[:, :, pl.ds(p_base, ppt)])
            else:
                def interp(g, _):
                    _pix, (wx, wy) = neighborhood(g)
                    w00 = (1.0 - wx) * (1.0 - wy)
                    w01 = wx * (1.0 - wy)
                    w10 = (1.0 - wx) * wy
                    w11 = wx * wy
                    for c in range(C):
                        o = ((g * C + c) * 4) * 16
                        r = o // 128
                        f00 = dest_v[r, pl.ds(o % 128, 16)]
                        f01 = dest_v[r, pl.ds(o % 128 + 16, 16)]
                        o2 = o + 32
                        f10 = dest_v[o2 // 128, pl.ds(o2 % 128, 16)]
                        f11 = dest_v[o2 // 128, pl.ds(o2 % 128 + 16, 16)]
                        v = f00 * w00 + f01 * w01 + f10 * w10 + f11 * w11
                        out_v[c, pl.ds(g * 16, 16)] = v
                    return 0

                lax.fori_loop(0, G, interp, 0)
                pltpu.sync_copy(out_v, out_hbm.at[:, pl.ds(p_base, ppt)])

    return sampler(feat_flat, xs, ys)


def _mlp_kernel(xm_ref, xf_ref, w1m_ref, w1f_ref, b1_ref, w2_ref, b2_ref,
                w3_ref, b3_ref, wf_ref, bf_ref, o_ref):
    dn = (((1,), (0,)), ((), ()))
    h = lax.dot_general(w1m_ref[...], xm_ref[...], dn,
                        preferred_element_type=jnp.float32)
    h += lax.dot_general(w1f_ref[...], xf_ref[...], dn,
                         preferred_element_type=jnp.float32)
    h = jnp.maximum(h + b1_ref[...], 0.0)
    h = jnp.maximum(lax.dot_general(w2_ref[...], h, dn,
                                    preferred_element_type=jnp.float32)
                    + b2_ref[...], 0.0)
    h = jnp.maximum(lax.dot_general(w3_ref[...], h, dn,
                                    preferred_element_type=jnp.float32)
                    + b3_ref[...], 0.0)
    o_ref[...] = lax.dot_general(wf_ref[...], h, dn,
                                 preferred_element_type=jnp.float32) \
        + bf_ref[...]


@functools.partial(jax.jit, static_argnames=("blk",))
def _mlp_pallas(params, xm, xf, blk=1024):
    W1, b1, W2, b2, W3, b3, Wf, bf = params
    Cf, M = xf.shape
    w1m = W1[:, :N_CLASS]
    w1f = W1[:, N_CLASS:]
    blk = min(blk, M)
    grid = (M // blk,)
    return pl.pallas_call(
        _mlp_kernel,
        grid=grid,
        in_specs=[
            pl.BlockSpec((N_CLASS, blk), lambda i: (0, i)),
            pl.BlockSpec((Cf, blk), lambda i: (0, i)),
            pl.BlockSpec((512, N_CLASS), lambda i: (0, 0)),
            pl.BlockSpec((512, Cf), lambda i: (0, 0)),
            pl.BlockSpec((512, 1), lambda i: (0, 0)),
            pl.BlockSpec((512, 512), lambda i: (0, 0)),
            pl.BlockSpec((512, 1), lambda i: (0, 0)),
            pl.BlockSpec((512, 512), lambda i: (0, 0)),
            pl.BlockSpec((512, 1), lambda i: (0, 0)),
            pl.BlockSpec((N_CLASS, 512), lambda i: (0, 0)),
            pl.BlockSpec((N_CLASS, 1), lambda i: (0, 0)),
        ],
        out_specs=pl.BlockSpec((N_CLASS, blk), lambda i: (0, i)),
        out_shape=jax.ShapeDtypeStruct((N_CLASS, M), jnp.float32),
    )(xm, xf, w1m, w1f, b1[:, None], W2, b2[:, None], W3, b3[:, None],
      Wf, bf[:, None])


def _stage(temp, feat, params, pts, sc_chunk):
    B, C, H, W = feat.shape
    N = pts.shape[1]
    xs = pts[..., 0].reshape(-1)
    ys = pts[..., 1].reshape(-1)
    if sc_chunk is not None:
        xf = _sc_point_sample(feat.reshape(-1), xs, ys,
                              C=C, H=H, W=W, N=N, c_chunk=sc_chunk)
    else:
        xf = jnp.transpose(_point_sample(feat, pts), (1, 0, 2)).reshape(C, B * N)
    th, tw = temp.shape[2], temp.shape[3]
    xm = _sc_point_sample_indirect(temp.reshape(-1), xs, ys,
                                   C=8, H=th, W=tw, N=N)
    out = _mlp_pallas(params, xm, xf)
    return jnp.transpose(out.reshape(N_CLASS, B, N), (1, 0, 2))


def kernel(refine, x0, x1, x2, x3, coarse, p3, p2, p1, p0, pr):
    key = jax.random.key(42)
    ks = jax.random.split(key, 10)
    temp1 = coarse
    pts1 = _sampling_points(jax.nn.softmax(temp1, axis=1), 512, ks[0], ks[1])
    rend1 = _stage(temp1, x3, p3, pts1, 8)
    temp2 = coarse
    pts2 = _sampling_points(jax.nn.softmax(temp2, axis=1), 512, ks[2], ks[3])
    rend2 = _stage(temp2, x2, p2, pts2, 8)
    temp3 = _upsample2x(temp2)
    pts3 = _sampling_points(jax.nn.softmax(temp3, axis=1), 2048, ks[4], ks[5])
    rend3 = _stage(temp3, x1, p1, pts3, 2)
    temp4 = _upsample2x(temp3)
    pts4 = _sampling_points(jax.nn.softmax(temp4, axis=1), 2048, ks[6], ks[7])
    rend4 = _stage(temp4, x0, p0, pts4, None)
    temp5 = _upsample2x(temp4)
    pts5 = _sampling_points(jax.nn.softmax(temp5, axis=1), 2048, ks[8], ks[9])
    rend5 = _stage(temp5, refine, pr, pts5, None)
    return (pts1, rend1, pts2, rend2, pts3, rend3, pts4, rend4, pts5, rend5)
